# baseline (speedup 1.0000x reference)
import functools

import jax
import jax.numpy as jnp
from jax import lax
from jax.experimental import pallas as pl
from jax.experimental.pallas import tpu as pltpu

_TN = 1024
_ME = 256
_RI = 8
_CJ = 64
_K = 8
_VMEM = 60 * 1024 * 1024


def _proj_kernel(x_ref, pos_ref, w1s_ref, w1d_ref, w1p_ref, a_ref, b_ref):
    pw = jnp.dot(pos_ref[...], w1p_ref[...], preferred_element_type=jnp.float32)
    a_ref[...] = jnp.dot(x_ref[...], w1s_ref[...],
                         preferred_element_type=jnp.float32) + pw
    b_ref[...] = jnp.dot(x_ref[...], w1d_ref[...],
                         preferred_element_type=jnp.float32) - pw


def _node_proj(x, pos8, w1s, w1d, w1p8):
    n, c = x.shape
    h = w1s.shape[1]
    grid = (n // _TN,)
    return pl.pallas_call(
        _proj_kernel,
        out_shape=(jax.ShapeDtypeStruct((n, h), jnp.float32),
                   jax.ShapeDtypeStruct((n, h), jnp.float32)),
        grid=grid,
        in_specs=[
            pl.BlockSpec((_TN, c), lambda i: (i, 0)),
            pl.BlockSpec((_TN, 8), lambda i: (i, 0)),
            pl.BlockSpec((c, h), lambda i: (0, 0)),
            pl.BlockSpec((c, h), lambda i: (0, 0)),
            pl.BlockSpec((8, h), lambda i: (0, 0)),
        ],
        out_specs=(pl.BlockSpec((_TN, h), lambda i: (i, 0)),
                   pl.BlockSpec((_TN, h), lambda i: (i, 0))),
        compiler_params=pltpu.CompilerParams(
            dimension_semantics=("parallel",),
            vmem_limit_bytes=_VMEM),
    )(x, pos8, w1s, w1d, w1p8)


def _score_kernel(s4_ref, d4_ref, a_ref, b_ref, b1_ref, w2rep_ref, b2_ref,
                  sb_ref, sc_ref, tile_ref, *, me, hdim):
    e0 = pl.program_id(0) * me
    p = hdim // 128
    stride = me + 1
    for mi in range(me):
        si = pl.multiple_of(s4_ref[e0 + mi], p)
        di = pl.multiple_of(d4_ref[e0 + mi], p)
        slab = a_ref[pl.ds(si, p), :] + b_ref[pl.ds(di, p), :]
        tile_ref[mi:mi + stride * p:stride, :] = slab
    chunks = [tile_ref[c * stride:c * stride + me, :] for c in range(p)]
    hh = jnp.concatenate(chunks, axis=-1)
    hh = jnp.maximum(hh + b1_ref[...], 0.0)
    sb = jnp.dot(hh, w2rep_ref[...], precision=jax.lax.Precision.HIGHEST,
                 preferred_element_type=jnp.float32) + b2_ref[...]
    sb_ref[...] = sb
    sc_ref[...] = sb[:, 0:1]


def _edge_scores(src4, dst4, a2, b2t, b1r, w2rep, b2c, e):
    hdim = b1r.shape[1]
    p = hdim // 128
    grid = (e // _ME,)
    spec = pltpu.PrefetchScalarGridSpec(
        num_scalar_prefetch=2,
        grid=grid,
        in_specs=[
            pl.BlockSpec(a2.shape, lambda i, s4, d4: (0, 0)),
            pl.BlockSpec(b2t.shape, lambda i, s4, d4: (0, 0)),
            pl.BlockSpec(b1r.shape, lambda i, s4, d4: (0, 0)),
            pl.BlockSpec(w2rep.shape, lambda i, s4, d4: (0, 0)),
            pl.BlockSpec(b2c.shape, lambda i, s4, d4: (0, 0)),
        ],
        out_specs=(pl.BlockSpec((_ME, 128), lambda i, s4, d4: (i, 0)),
                   pl.BlockSpec((_ME, 1), lambda i, s4, d4: (i, 0))),
        scratch_shapes=[pltpu.VMEM(((_ME + 1) * p, 128), jnp.float32)],
    )
    return pl.pallas_call(
        functools.partial(_score_kernel, me=_ME, hdim=hdim),
        grid_spec=spec,
        out_shape=(jax.ShapeDtypeStruct((e, 128), jnp.float32),
                   jax.ShapeDtypeStruct((e, 1), jnp.float32)),
        compiler_params=pltpu.CompilerParams(
            dimension_semantics=("parallel",),
            vmem_limit_bytes=_VMEM),
    )(src4, dst4, a2, b2t, b1r, w2rep, b2c)


def _topk_kernel(si_ref, di_ref, sb_ref, db_ref, mask_ref, *, k, e, ri, cj):
    si = si_ref[...]
    di = di_ref[...]
    sib = [jnp.broadcast_to(si[r:r + 1, :], (8, 128)) for r in range(ri)]
    dib = [jnp.broadcast_to(di[r:r + 1, :], (8, 128)) for r in range(ri)]

    def chunk(c, accs):
        base = pl.multiple_of(c * cj, cj)
        sj = sb_ref[pl.ds(base, cj), :]
        dj = db_ref[pl.ds(base, cj), :]
        new = list(accs)
        for u in range(cj // 8):
            sju = sj[u * 8:(u + 1) * 8, :]
            dju = dj[u * 8:(u + 1) * 8, :]
            for r in range(ri):
                beats = jnp.where(sju > sib[r], 1, 0)
                new[r] = new[r] + jnp.where(dju == dib[r], beats, 0)
        return tuple(new)

    acc0 = tuple(jnp.zeros((8, 128), jnp.int32) for _ in range(ri))
    accs = lax.fori_loop(0, e // cj, chunk, acc0)
    ranks = jnp.concatenate(
        [jnp.sum(a, axis=0, keepdims=True) for a in accs], axis=0)
    mask_ref[...] = (ranks < k).astype(jnp.int32)


def _topk_mask(s2d, d2d, s_b, d_b, e):
    nrow = e // 128
    nstep = nrow // _RI
    grid = (2, nstep // 2)
    imap = lambda c, t: (c * (nstep // 2) + t, 0)
    return pl.pallas_call(
        functools.partial(_topk_kernel, k=_K, e=e, ri=_RI, cj=_CJ),
        out_shape=jax.ShapeDtypeStruct((nrow, 128), jnp.int32),
        grid=grid,
        in_specs=[
            pl.BlockSpec((_RI, 128), imap),
            pl.BlockSpec((_RI, 128), imap),
            pl.BlockSpec((e, 128), lambda c, t: (0, 0)),
            pl.BlockSpec((e, 128), lambda c, t: (0, 0)),
        ],
        out_specs=pl.BlockSpec((_RI, 128), imap),
        compiler_params=pltpu.CompilerParams(
            dimension_semantics=("parallel", "arbitrary"),
            vmem_limit_bytes=_VMEM),
    )(s2d, d2d, s_b, d_b)


def kernel(x, pos, full_edge_index, w1, b1, w2, b2):
    src = full_edge_index[0].astype(jnp.int32)
    dst = full_edge_index[1].astype(jnp.int32)
    e = src.shape[0]
    n, c = x.shape
    h = w1.shape[1]
    p = h // 128

    xf = x.astype(jnp.float32)
    posf = pos.astype(jnp.float32)
    w1f = w1.astype(jnp.float32)
    pos8 = jnp.pad(posf, ((0, 0), (0, 8 - posf.shape[1])))
    w1s = w1f[0:c]
    w1d = w1f[c:2 * c]
    w1p8 = jnp.pad(w1f[2 * c:], ((0, 8 - (w1f.shape[0] - 2 * c)), (0, 0)))

    a, b = _node_proj(xf, pos8, w1s, w1d, w1p8)
    a2 = a.reshape(n * p, 128)
    b2t = b.reshape(n * p, 128)

    b1r = b1.astype(jnp.float32).reshape(1, h)
    w2rep = jnp.broadcast_to(w2.astype(jnp.float32).reshape(h, 1), (h, 128))
    b2c = b2.astype(jnp.float32).reshape(1, 1)
    src4 = src * p
    dst4 = dst * p

    s_b, scores_col = _edge_scores(src4, dst4, a2, b2t, b1r, w2rep, b2c, e)

    s2d = scores_col.reshape(e // 128, 128)
    d2d = dst.reshape(e // 128, 128)
    d_b = jnp.broadcast_to(dst.reshape(e, 1), (e, 128))
    mask2d = _topk_mask(s2d, d2d, s_b, d_b, e)

    mask = mask2d.reshape(e) > 0
    scores = scores_col[:, 0]
    return full_edge_index, mask, scores


# speedup vs baseline: 4.1800x; 4.1800x over previous
import functools

import jax
import jax.numpy as jnp
from jax import lax
from jax.experimental import pallas as pl
from jax.experimental.pallas import tpu as pltpu

_TN = 1024
_ME = 256
_K = 8
_VMEM = 60 * 1024 * 1024


def _proj_kernel(x_ref, pos_ref, w1s_ref, w1d_ref, w1p_ref, a_ref, b_ref):
    pw = jnp.dot(pos_ref[...], w1p_ref[...], preferred_element_type=jnp.float32)
    a_ref[...] = jnp.dot(x_ref[...], w1s_ref[...],
                         preferred_element_type=jnp.float32) + pw
    b_ref[...] = jnp.dot(x_ref[...], w1d_ref[...],
                         preferred_element_type=jnp.float32) - pw


def _node_proj(x, pos8, w1s, w1d, w1p8):
    n, c = x.shape
    h = w1s.shape[1]
    grid = (n // _TN,)
    return pl.pallas_call(
        _proj_kernel,
        out_shape=(jax.ShapeDtypeStruct((n, h), jnp.float32),
                   jax.ShapeDtypeStruct((n, h), jnp.float32)),
        grid=grid,
        in_specs=[
            pl.BlockSpec((_TN, c), lambda i: (i, 0)),
            pl.BlockSpec((_TN, 8), lambda i: (i, 0)),
            pl.BlockSpec((c, h), lambda i: (0, 0)),
            pl.BlockSpec((c, h), lambda i: (0, 0)),
            pl.BlockSpec((8, h), lambda i: (0, 0)),
        ],
        out_specs=(pl.BlockSpec((_TN, h), lambda i: (i, 0)),
                   pl.BlockSpec((_TN, h), lambda i: (i, 0))),
        compiler_params=pltpu.CompilerParams(
            dimension_semantics=("parallel",),
            vmem_limit_bytes=_VMEM),
    )(x, pos8, w1s, w1d, w1p8)


def _score_kernel(s4_ref, d4_ref, a_ref, b_ref, b1_ref, w2_ref, b2_ref,
                  sc_ref, tile_ref, *, me, hdim):
    e0 = pl.program_id(0) * me
    p = hdim // 128
    stride = me + 1
    for mi in range(me):
        si = pl.multiple_of(s4_ref[e0 + mi], p)
        di = pl.multiple_of(d4_ref[e0 + mi], p)
        slab = a_ref[pl.ds(si, p), :] + b_ref[pl.ds(di, p), :]
        tile_ref[mi:mi + stride * p:stride, :] = slab
    chunks = [tile_ref[c * stride:c * stride + me, :] for c in range(p)]
    hh = jnp.concatenate(chunks, axis=-1)
    hh = jnp.maximum(hh + b1_ref[...], 0.0)
    sc_ref[...] = jnp.sum(hh * w2_ref[...], axis=1,
                          keepdims=True) + b2_ref[...]


def _edge_scores(src4, dst4, a2, b2t, b1r, w2r, b2c, e):
    hdim = b1r.shape[1]
    p = hdim // 128
    grid = (e // _ME,)
    spec = pltpu.PrefetchScalarGridSpec(
        num_scalar_prefetch=2,
        grid=grid,
        in_specs=[
            pl.BlockSpec(a2.shape, lambda i, s4, d4: (0, 0)),
            pl.BlockSpec(b2t.shape, lambda i, s4, d4: (0, 0)),
            pl.BlockSpec(b1r.shape, lambda i, s4, d4: (0, 0)),
            pl.BlockSpec(w2r.shape, lambda i, s4, d4: (0, 0)),
            pl.BlockSpec(b2c.shape, lambda i, s4, d4: (0, 0)),
        ],
        out_specs=pl.BlockSpec((_ME, 1), lambda i, s4, d4: (i, 0)),
        scratch_shapes=[pltpu.VMEM(((_ME + 1) * p, 128), jnp.float32)],
    )
    return pl.pallas_call(
        functools.partial(_score_kernel, me=_ME, hdim=hdim),
        grid_spec=spec,
        out_shape=jax.ShapeDtypeStruct((e, 1), jnp.float32),
        compiler_params=pltpu.CompilerParams(
            dimension_semantics=("parallel",),
            vmem_limit_bytes=_VMEM),
    )(src4, dst4, a2, b2t, b1r, w2r, b2c)


def _bitonic_passes(refs, nkeys, pos, lanei, rows, n):
    def lt_fn(a, b):
        lt = a[0] < b[0]
        for t in range(1, nkeys):
            eq = a[0] == b[0]
            for u in range(1, t):
                eq = jnp.logical_and(eq, a[u] == b[u])
            lt = jnp.logical_or(lt, jnp.logical_and(eq, a[t] < b[t]))
        return lt

    def lane_passes(size, log_hi):
        asc = (pos & size) == 0

        def body(t, carry):
            stride = jnp.int32(1) << (log_hi - t)
            low = (lanei & stride) == 0
            cur = [r[...] for r in refs]
            prt = [jnp.where(low,
                             pltpu.roll(x, 128 - stride, axis=1),
                             pltpu.roll(x, stride, axis=1))
                   for x in cur]
            lt = lt_fn(cur, prt)
            takemin = low == asc
            choose_self = lt == takemin
            for r, x, px in zip(refs, cur, prt):
                r[...] = jnp.where(choose_self, x, px)
            return carry

        lax.fori_loop(0, log_hi + 1, body, 0)

    for sl in range(1, n.bit_length()):
        size = 1 << sl
        for st in range(sl - 1, 6, -1):
            stride = 1 << st
            rs = stride // 128
            g = rows // (2 * rs)
            sh = (g, 2, rs, 128)
            sp = [r[...].reshape(sh) for r in refs]
            pa = pos.reshape(sh)[:, 0]
            a = [x[:, 0] for x in sp]
            b = [x[:, 1] for x in sp]
            lt = lt_fn(a, b)
            asc = (pa & size) == 0
            swap = jnp.logical_xor(lt, asc)
            for r, x, y in zip(refs, a, b):
                na = jnp.where(swap, y, x)
                nb = jnp.where(swap, x, y)
                r[...] = jnp.stack([na, nb], axis=1).reshape(rows, 128)
        lane_passes(size, min(sl - 1, 6))


def _sort_topk_kernel(s_ref, d_ref, mask_ref, ka_ref, pb_ref, *, e, k):
    rows = e // 128
    rowi = lax.broadcasted_iota(jnp.int32, (rows, 128), 0)
    lanei = lax.broadcasted_iota(jnp.int32, (rows, 128), 1)
    pos = rowi * 128 + lanei

    bits = pltpu.bitcast(s_ref[...], jnp.int32)
    oi = bits ^ lax.shift_right_logical(
        lax.shift_right_arithmetic(bits, 31), 1)
    ka_ref[...] = jnp.invert(oi)
    pb_ref[...] = (d_ref[...] << 15) | pos

    _bitonic_passes([ka_ref, pb_ref], 2, pos, lanei, rows, e)

    ka_ref[...] = ((pb_ref[...] >> 15) << 15) | pos
    _bitonic_passes([ka_ref, pb_ref], 1, pos, lanei, rows, e)

    d_s = ka_ref[...] >> 15
    r1 = pltpu.roll(d_s, k, axis=1)
    r2 = pltpu.roll(pltpu.roll(d_s, 1, axis=0), k, axis=1)
    dm8 = jnp.where(lanei >= k, r1, r2)
    keep = jnp.logical_or(pos < k, d_s != dm8)

    ka_ref[...] = ((pb_ref[...] & (e - 1)) << 1) | keep.astype(jnp.int32)
    _bitonic_passes([ka_ref], 1, pos, lanei, rows, e)
    mask_ref[...] = ka_ref[...] & 1


def _topk_mask(s2d, d2d, e):
    nrow = e // 128
    return pl.pallas_call(
        functools.partial(_sort_topk_kernel, e=e, k=_K),
        out_shape=jax.ShapeDtypeStruct((nrow, 128), jnp.int32),
        grid=(1,),
        in_specs=[
            pl.BlockSpec((nrow, 128), lambda i: (0, 0)),
            pl.BlockSpec((nrow, 128), lambda i: (0, 0)),
        ],
        out_specs=pl.BlockSpec((nrow, 128), lambda i: (0, 0)),
        scratch_shapes=[pltpu.VMEM((nrow, 128), jnp.int32),
                        pltpu.VMEM((nrow, 128), jnp.int32)],
        compiler_params=pltpu.CompilerParams(
            dimension_semantics=("arbitrary",),
            vmem_limit_bytes=_VMEM),
    )(s2d, d2d)


def kernel(x, pos, full_edge_index, w1, b1, w2, b2):
    src = full_edge_index[0].astype(jnp.int32)
    dst = full_edge_index[1].astype(jnp.int32)
    e = src.shape[0]
    n, c = x.shape
    h = w1.shape[1]
    p = h // 128

    xf = x.astype(jnp.float32)
    posf = pos.astype(jnp.float32)
    w1f = w1.astype(jnp.float32)
    pos8 = jnp.pad(posf, ((0, 0), (0, 8 - posf.shape[1])))
    w1s = w1f[0:c]
    w1d = w1f[c:2 * c]
    w1p8 = jnp.pad(w1f[2 * c:], ((0, 8 - (w1f.shape[0] - 2 * c)), (0, 0)))

    a, b = _node_proj(xf, pos8, w1s, w1d, w1p8)
    a2 = a.reshape(n * p, 128)
    b2t = b.reshape(n * p, 128)

    b1r = b1.astype(jnp.float32).reshape(1, h)
    w2r = w2.astype(jnp.float32).reshape(1, h)
    b2c = b2.astype(jnp.float32).reshape(1, 1)
    src4 = src * p
    dst4 = dst * p

    scores_col = _edge_scores(src4, dst4, a2, b2t, b1r, w2r, b2c, e)

    s2d = scores_col.reshape(e // 128, 128)
    d2d = dst.reshape(e // 128, 128)
    mask2d = _topk_mask(s2d, d2d, e)

    mask = mask2d.reshape(e) > 0
    scores = scores_col[:, 0]
    return full_edge_index, mask, scores


# speedup vs baseline: 4.4097x; 1.0550x over previous
import functools

import jax
import jax.numpy as jnp
from jax import lax
from jax.experimental import pallas as pl
from jax.experimental.pallas import tpu as pltpu

_TN = 1024
_ME = 512
_K = 8
_VMEM = 60 * 1024 * 1024


def _proj_kernel(x_ref, pos_ref, wsd_ref, w1p_ref, a_ref, b_ref):
    h = a_ref.shape[1]
    pw = jnp.dot(pos_ref[...], w1p_ref[...], preferred_element_type=jnp.float32)
    ab = jnp.dot(x_ref[...], wsd_ref[...], preferred_element_type=jnp.float32)
    a_ref[...] = ab[:, :h] + pw
    b_ref[...] = ab[:, h:] - pw


def _node_proj(x, pos8, wsd, w1p8):
    n, c = x.shape
    h = wsd.shape[1] // 2
    grid = (n // _TN,)
    return pl.pallas_call(
        _proj_kernel,
        out_shape=(jax.ShapeDtypeStruct((n, h), jnp.float32),
                   jax.ShapeDtypeStruct((n, h), jnp.float32)),
        grid=grid,
        in_specs=[
            pl.BlockSpec((_TN, c), lambda i: (i, 0)),
            pl.BlockSpec((_TN, 8), lambda i: (i, 0)),
            pl.BlockSpec((c, 2 * h), lambda i: (0, 0)),
            pl.BlockSpec((8, h), lambda i: (0, 0)),
        ],
        out_specs=(pl.BlockSpec((_TN, h), lambda i: (i, 0)),
                   pl.BlockSpec((_TN, h), lambda i: (i, 0))),
        compiler_params=pltpu.CompilerParams(
            dimension_semantics=("parallel",),
            vmem_limit_bytes=_VMEM),
    )(x, pos8, wsd, w1p8)


def _score_kernel(s4_ref, d4_ref, a_ref, b_ref, b1_ref, w2_ref, b2_ref,
                  sc_ref, tile_ref, *, me, hdim):
    e0 = pl.program_id(0) * me
    p = hdim // 128
    stride = me + 1
    for mi in range(me):
        si = pl.multiple_of(s4_ref[e0 + mi], p)
        di = pl.multiple_of(d4_ref[e0 + mi], p)
        slab = a_ref[pl.ds(si, p), :] + b_ref[pl.ds(di, p), :]
        tile_ref[mi:mi + stride * p:stride, :] = slab
    chunks = [tile_ref[c * stride:c * stride + me, :] for c in range(p)]
    hh = jnp.concatenate(chunks, axis=-1)
    hh = jnp.maximum(hh + b1_ref[...], 0.0)
    sc_ref[...] = jnp.sum(hh * w2_ref[...], axis=1,
                          keepdims=True) + b2_ref[...]


def _edge_scores(src4, dst4, a2, b2t, b1r, w2r, b2c, e):
    hdim = b1r.shape[1]
    p = hdim // 128
    grid = (e // _ME,)
    spec = pltpu.PrefetchScalarGridSpec(
        num_scalar_prefetch=2,
        grid=grid,
        in_specs=[
            pl.BlockSpec(a2.shape, lambda i, s4, d4: (0, 0)),
            pl.BlockSpec(b2t.shape, lambda i, s4, d4: (0, 0)),
            pl.BlockSpec(b1r.shape, lambda i, s4, d4: (0, 0)),
            pl.BlockSpec(w2r.shape, lambda i, s4, d4: (0, 0)),
            pl.BlockSpec(b2c.shape, lambda i, s4, d4: (0, 0)),
        ],
        out_specs=pl.BlockSpec((_ME, 1), lambda i, s4, d4: (i, 0)),
        scratch_shapes=[pltpu.VMEM(((_ME + 1) * p, 128), jnp.float32)],
    )
    return pl.pallas_call(
        functools.partial(_score_kernel, me=_ME, hdim=hdim),
        grid_spec=spec,
        out_shape=jax.ShapeDtypeStruct((e, 1), jnp.float32),
        compiler_params=pltpu.CompilerParams(
            dimension_semantics=("parallel",),
            vmem_limit_bytes=_VMEM),
    )(src4, dst4, a2, b2t, b1r, w2r, b2c)


def _bitonic_passes(refs, nkeys, pos, lanei, rows, n):
    def lt_fn(a, b):
        lt = a[0] < b[0]
        for t in range(1, nkeys):
            eq = a[0] == b[0]
            for u in range(1, t):
                eq = jnp.logical_and(eq, a[u] == b[u])
            lt = jnp.logical_or(lt, jnp.logical_and(eq, a[t] < b[t]))
        return lt

    def lane_passes(size, log_hi):
        asc = (pos & size) == 0

        def body(t, carry):
            stride = jnp.int32(1) << (log_hi - t)
            low = (lanei & stride) == 0
            cur = [r[...] for r in refs]
            prt = [jnp.where(low,
                             pltpu.roll(x, 128 - stride, axis=1),
                             pltpu.roll(x, stride, axis=1))
                   for x in cur]
            lt = lt_fn(cur, prt)
            takemin = low == asc
            choose_self = lt == takemin
            for r, x, px in zip(refs, cur, prt):
                r[...] = jnp.where(choose_self, x, px)
            return carry

        lax.fori_loop(0, log_hi + 1, body, 0)

    for sl in range(1, n.bit_length()):
        size = 1 << sl
        for st in range(sl - 1, 6, -1):
            stride = 1 << st
            rs = stride // 128
            g = rows // (2 * rs)
            sh = (g, 2, rs, 128)
            sp = [r[...].reshape(sh) for r in refs]
            pa = pos.reshape(sh)[:, 0]
            a = [x[:, 0] for x in sp]
            b = [x[:, 1] for x in sp]
            lt = lt_fn(a, b)
            asc = (pa & size) == 0
            swap = jnp.logical_xor(lt, asc)
            for r, x, y in zip(refs, a, b):
                na = jnp.where(swap, y, x)
                nb = jnp.where(swap, x, y)
                r[...] = jnp.stack([na, nb], axis=1).reshape(rows, 128)
        lane_passes(size, min(sl - 1, 6))


def _sort_topk_kernel(s_ref, d_ref, mask_ref, ka_ref, pb_ref, *, e, k):
    rows = e // 128
    rowi = lax.broadcasted_iota(jnp.int32, (rows, 128), 0)
    lanei = lax.broadcasted_iota(jnp.int32, (rows, 128), 1)
    pos = rowi * 128 + lanei

    bits = pltpu.bitcast(s_ref[...], jnp.int32)
    oi = bits ^ lax.shift_right_logical(
        lax.shift_right_arithmetic(bits, 31), 1)
    ka_ref[...] = jnp.invert(oi)
    pb_ref[...] = (d_ref[...] << 15) | pos

    _bitonic_passes([ka_ref, pb_ref], 2, pos, lanei, rows, e)

    ka_ref[...] = ((pb_ref[...] >> 15) << 15) | pos
    _bitonic_passes([ka_ref, pb_ref], 1, pos, lanei, rows, e)

    d_s = ka_ref[...] >> 15
    r1 = pltpu.roll(d_s, k, axis=1)
    r2 = pltpu.roll(pltpu.roll(d_s, 1, axis=0), k, axis=1)
    dm8 = jnp.where(lanei >= k, r1, r2)
    keep = jnp.logical_or(pos < k, d_s != dm8)

    ka_ref[...] = ((pb_ref[...] & (e - 1)) << 1) | keep.astype(jnp.int32)
    _bitonic_passes([ka_ref], 1, pos, lanei, rows, e)
    mask_ref[...] = ka_ref[...] & 1


def _topk_mask(s2d, d2d, e):
    nrow = e // 128
    return pl.pallas_call(
        functools.partial(_sort_topk_kernel, e=e, k=_K),
        out_shape=jax.ShapeDtypeStruct((nrow, 128), jnp.int32),
        grid=(1,),
        in_specs=[
            pl.BlockSpec((nrow, 128), lambda i: (0, 0)),
            pl.BlockSpec((nrow, 128), lambda i: (0, 0)),
        ],
        out_specs=pl.BlockSpec((nrow, 128), lambda i: (0, 0)),
        scratch_shapes=[pltpu.VMEM((nrow, 128), jnp.int32),
                        pltpu.VMEM((nrow, 128), jnp.int32)],
        compiler_params=pltpu.CompilerParams(
            dimension_semantics=("arbitrary",),
            vmem_limit_bytes=_VMEM),
    )(s2d, d2d)


def kernel(x, pos, full_edge_index, w1, b1, w2, b2):
    src = full_edge_index[0].astype(jnp.int32)
    dst = full_edge_index[1].astype(jnp.int32)
    e = src.shape[0]
    n, c = x.shape
    h = w1.shape[1]
    p = h // 128

    xf = x.astype(jnp.float32)
    posf = pos.astype(jnp.float32)
    w1f = w1.astype(jnp.float32)
    pos8 = jnp.pad(posf, ((0, 0), (0, 8 - posf.shape[1])))
    w1s = w1f[0:c]
    w1d = w1f[c:2 * c]
    w1p8 = jnp.pad(w1f[2 * c:], ((0, 8 - (w1f.shape[0] - 2 * c)), (0, 0)))

    wsd = jnp.concatenate([w1s, w1d], axis=1)
    a, b = _node_proj(xf, pos8, wsd, w1p8)
    a2 = a.reshape(n * p, 128)
    b2t = b.reshape(n * p, 128)

    b1r = b1.astype(jnp.float32).reshape(1, h)
    w2r = w2.astype(jnp.float32).reshape(1, h)
    b2c = b2.astype(jnp.float32).reshape(1, 1)
    src4 = src * p
    dst4 = dst * p

    scores_col = _edge_scores(src4, dst4, a2, b2t, b1r, w2r, b2c, e)

    s2d = scores_col.reshape(e // 128, 128)
    d2d = dst.reshape(e // 128, 128)
    mask2d = _topk_mask(s2d, d2d, e)

    mask = mask2d.reshape(e) > 0
    scores = scores_col[:, 0]
    return full_edge_index, mask, scores
